# baseline (speedup 1.0000x reference)
import functools

import jax
import jax.numpy as jnp
from jax import lax
from jax.experimental import pallas as pl
from jax.experimental.pallas import tpu as pltpu
from jax.experimental.pallas import tpu_sc as plsc

_NUM_TOK = 8192
_IN_DIM = 4096
_NUM_EXPERTS = 64
_D_E = 64
_TOP_K = 8
_BLK = 1024

_NW = 32
_SC_ROWS = _NUM_TOK // _NW
_SC_GROUPS = _SC_ROWS // 16


def _gates_block(tau_ref, h_ref, w_ref, ee_ref, fg_ref):
    f32 = jnp.float32
    hp = jax.lax.dot_general(
        h_ref[...], w_ref[...], (((1,), (1,)), ((), ())),
        preferred_element_type=f32, precision=jax.lax.Precision.DEFAULT)
    hpt = hp.T
    nrm = jnp.sqrt(jnp.sum(hpt * hpt, axis=0, keepdims=True))
    hnt = hpt / jnp.maximum(nrm, 1e-12)
    ee = ee_ref[...]
    een = ee / jnp.maximum(
        jnp.sqrt(jnp.sum(ee * ee, axis=-1, keepdims=True)), 1e-12)
    scores = jax.lax.dot_general(
        een, hnt, (((1,), (0,)), ((), ())),
        preferred_element_type=f32, precision=jax.lax.Precision.DEFAULT)
    x = scores / tau_ref[0]
    m = jnp.max(x, axis=0, keepdims=True)
    ex = jnp.exp(x - m)
    fg = ex / jnp.sum(ex, axis=0, keepdims=True)
    fg_ref[...] = fg.T


def _tc_gates(h, W, expert_embeddings, tau):
    grid = (_NUM_TOK // _BLK,)
    return pl.pallas_call(
        _gates_block,
        grid=grid,
        in_specs=[
            pl.BlockSpec(memory_space=pltpu.SMEM),
            pl.BlockSpec((_BLK, _IN_DIM), lambda i: (i, 0)),
            pl.BlockSpec((_D_E, _IN_DIM), lambda i: (0, 0)),
            pl.BlockSpec((_NUM_EXPERTS, _D_E), lambda i: (0, 0)),
        ],
        out_specs=pl.BlockSpec((_BLK, _NUM_EXPERTS), lambda i: (i, 0)),
        out_shape=jax.ShapeDtypeStruct((_NUM_TOK, _NUM_EXPERTS), jnp.float32),
        compiler_params=pltpu.CompilerParams(
            dimension_semantics=("arbitrary",),
        ),
    )(jnp.reshape(tau, (1,)), h, W, expert_embeddings)


def _merge(va, ia, vb, ib):
    upd = (vb > va) | ((vb == va) & (ib < ia))
    return jnp.where(upd, vb, va), jnp.where(upd, ib, ia)


def _sc_route_body(fg_hbm, sg_hbm, idx_hbm, fg_v, sg_v, idx_v):
    i32 = jnp.int32
    f32 = jnp.float32
    wid = lax.axis_index("s") * 2 + lax.axis_index("c")
    base = wid * _SC_ROWS
    pltpu.sync_copy(fg_hbm.at[pl.ds(base * _NUM_EXPERTS,
                                    _SC_ROWS * _NUM_EXPERTS)], fg_v)
    lanes = lax.iota(i32, 16)
    neg1 = jnp.full((16,), -1.0, f32)
    zero16 = jnp.zeros((16,), f32)

    def group_body(g, carry):
        rows64 = (g * 16 + lanes) * _NUM_EXPERTS
        rows8 = (g * 16 + lanes) * _TOP_K
        picks = []
        for _ in range(_TOP_K):
            accs = []
            for j in range(4):
                v = plsc.load_gather(fg_v, [rows64 + j])
                i = jnp.full((16,), j, i32)
                for e in range(j + 4, _NUM_EXPERTS, 4):
                    gv = plsc.load_gather(fg_v, [rows64 + e])
                    upd = gv > v
                    v = jnp.where(upd, gv, v)
                    i = jnp.where(upd, e, i)
                accs.append((v, i))
            va, ia = _merge(*accs[0], *accs[1])
            vb, ib = _merge(*accs[2], *accs[3])
            vm, im = _merge(va, ia, vb, ib)
            picks.append((vm, im))
            plsc.store_scatter(fg_v, [rows64 + im], neg1)
        v0 = picks[0][0]
        evs = [jnp.exp(v - v0) for v, _ in picks]
        z = evs[0]
        for t in evs[1:]:
            z = z + t
        for j in range(16 * _NUM_EXPERTS // 16):
            sg_v[pl.ds(g * (16 * _NUM_EXPERTS) + j * 16, 16)] = zero16
        for k in range(_TOP_K):
            vm, im = picks[k]
            plsc.store_scatter(sg_v, [rows64 + im], evs[k] / z)
            plsc.store_scatter(idx_v, [rows8 + k], im)
        return carry

    lax.fori_loop(0, _SC_GROUPS, group_body, 0)
    pltpu.sync_copy(sg_v, sg_hbm.at[pl.ds(base * _NUM_EXPERTS,
                                          _SC_ROWS * _NUM_EXPERTS)])
    pltpu.sync_copy(idx_v, idx_hbm.at[pl.ds(base * _TOP_K,
                                            _SC_ROWS * _TOP_K)])


@functools.cache
def _sc_route():
    return pl.kernel(
        _sc_route_body,
        out_type=[
            jax.ShapeDtypeStruct((_NUM_TOK * _NUM_EXPERTS,), jnp.float32),
            jax.ShapeDtypeStruct((_NUM_TOK * _TOP_K,), jnp.int32),
        ],
        mesh=plsc.VectorSubcoreMesh(core_axis_name="c", subcore_axis_name="s"),
        scratch_types=[
            pltpu.VMEM((_SC_ROWS * _NUM_EXPERTS,), jnp.float32),
            pltpu.VMEM((_SC_ROWS * _NUM_EXPERTS,), jnp.float32),
            pltpu.VMEM((_SC_ROWS * _TOP_K,), jnp.int32),
        ],
        compiler_params=pltpu.CompilerParams(needs_layout_passes=False),
    )


@jax.jit
def _router(h, W, expert_embeddings, tau):
    fg = _tc_gates(h, W, expert_embeddings, tau)
    sg_flat, idx_flat = _sc_route()(jnp.reshape(fg, (-1,)))
    sg = jnp.reshape(sg_flat, (_NUM_TOK, _NUM_EXPERTS))
    idx = jnp.reshape(idx_flat, (_NUM_TOK, _TOP_K))
    return sg, idx, fg


def kernel(h, W, expert_embeddings, tau):
    return _router(h, W, expert_embeddings, tau)


# speedup vs baseline: 1.4368x; 1.4368x over previous
import functools

import jax
import jax.numpy as jnp
from jax import lax
from jax.experimental import pallas as pl
from jax.experimental.pallas import tpu as pltpu
from jax.experimental.pallas import tpu_sc as plsc

_NUM_TOK = 8192
_IN_DIM = 4096
_NUM_EXPERTS = 64
_D_E = 64
_TOP_K = 8
_BLK = 1024

_NW = 32
_SC_ROWS = _NUM_TOK // _NW
_SC_GROUPS = _SC_ROWS // 16


def _gates_block(tau_ref, h_ref, w_ref, ee_ref, fg_ref):
    f32 = jnp.float32
    hp = jax.lax.dot_general(
        h_ref[...], w_ref[...], (((1,), (1,)), ((), ())),
        preferred_element_type=f32, precision=jax.lax.Precision.DEFAULT)
    hpt = hp.T
    nrm = jnp.sqrt(jnp.sum(hpt * hpt, axis=0, keepdims=True))
    hnt = hpt / jnp.maximum(nrm, 1e-12)
    ee = ee_ref[...]
    een = ee / jnp.maximum(
        jnp.sqrt(jnp.sum(ee * ee, axis=-1, keepdims=True)), 1e-12)
    scores = jax.lax.dot_general(
        een, hnt, (((1,), (0,)), ((), ())),
        preferred_element_type=f32, precision=jax.lax.Precision.DEFAULT)
    x = scores / tau_ref[0]
    m = jnp.max(x, axis=0, keepdims=True)
    ex = jnp.exp(x - m)
    fg = ex / jnp.sum(ex, axis=0, keepdims=True)
    fg_ref[...] = fg.T


def _tc_gates(h, W, expert_embeddings, tau):
    grid = (_NUM_TOK // _BLK,)
    return pl.pallas_call(
        _gates_block,
        grid=grid,
        in_specs=[
            pl.BlockSpec(memory_space=pltpu.SMEM),
            pl.BlockSpec((_BLK, _IN_DIM), lambda i: (i, 0)),
            pl.BlockSpec((_D_E, _IN_DIM), lambda i: (0, 0)),
            pl.BlockSpec((_NUM_EXPERTS, _D_E), lambda i: (0, 0)),
        ],
        out_specs=pl.BlockSpec((_BLK, _NUM_EXPERTS), lambda i: (i, 0)),
        out_shape=jax.ShapeDtypeStruct((_NUM_TOK, _NUM_EXPERTS), jnp.float32),
        compiler_params=pltpu.CompilerParams(
            dimension_semantics=("arbitrary",),
        ),
    )(jnp.reshape(tau, (1,)), h, W, expert_embeddings)


def _merge(va, ia, vb, ib):
    upd = (vb > va) | ((vb == va) & (ib < ia))
    return jnp.where(upd, vb, va), jnp.where(upd, ib, ia)


def _sc_route_body(fg_hbm, sg_hbm, idx_hbm, fg_v, sg_v, idx_v):
    i32 = jnp.int32
    f32 = jnp.float32
    wid = lax.axis_index("s") * 2 + lax.axis_index("c")
    base = wid * _SC_ROWS
    pltpu.sync_copy(fg_hbm.at[pl.ds(base * _NUM_EXPERTS,
                                    _SC_ROWS * _NUM_EXPERTS)], fg_v)
    lanes = lax.iota(i32, 16)
    zero16 = jnp.zeros((16,), f32)
    msk8 = lanes < _TOP_K

    def pmerge(a, b):
        ak, av = a
        bkr = lax.rev(b[0], (0,))
        bvr = lax.rev(b[1], (0,))
        ta = ak >= bkr
        hk = jnp.where(ta, ak, bkr)
        hv = jnp.where(ta, av, bvr)
        return plsc.sort_key_val(hk, hv, descending=True)

    def row_body(r, carry):
        off = r * _NUM_EXPERTS
        chunks = []
        for c in range(_NUM_EXPERTS // 16):
            keys = fg_v[pl.ds(off + c * 16, 16)]
            chunks.append(
                plsc.sort_key_val(keys, lanes + c * 16, descending=True))
        k16, v16 = pmerge(pmerge(chunks[0], chunks[1]),
                          pmerge(chunks[2], chunks[3]))
        ev = jnp.where(msk8, jnp.exp(k16), 0.0)
        nt = ev / jnp.sum(ev)
        for c in range(_NUM_EXPERTS // 16):
            sg_v[pl.ds(off + c * 16, 16)] = zero16
        plsc.store_scatter(sg_v, [off + v16], nt, mask=msk8)
        plsc.store_scatter(idx_v, [r * _TOP_K + lanes], v16, mask=msk8)
        return carry

    lax.fori_loop(0, _SC_ROWS, row_body, 0, unroll=4)
    pltpu.sync_copy(sg_v, sg_hbm.at[pl.ds(base * _NUM_EXPERTS,
                                          _SC_ROWS * _NUM_EXPERTS)])
    pltpu.sync_copy(idx_v, idx_hbm.at[pl.ds(base * _TOP_K,
                                            _SC_ROWS * _TOP_K)])


@functools.cache
def _sc_route():
    return pl.kernel(
        _sc_route_body,
        out_type=[
            jax.ShapeDtypeStruct((_NUM_TOK * _NUM_EXPERTS,), jnp.float32),
            jax.ShapeDtypeStruct((_NUM_TOK * _TOP_K,), jnp.int32),
        ],
        mesh=plsc.VectorSubcoreMesh(core_axis_name="c", subcore_axis_name="s"),
        scratch_types=[
            pltpu.VMEM((_SC_ROWS * _NUM_EXPERTS,), jnp.float32),
            pltpu.VMEM((_SC_ROWS * _NUM_EXPERTS,), jnp.float32),
            pltpu.VMEM((_SC_ROWS * _TOP_K,), jnp.int32),
        ],
        compiler_params=pltpu.CompilerParams(needs_layout_passes=False),
    )


@jax.jit
def _router(h, W, expert_embeddings, tau):
    fg = _tc_gates(h, W, expert_embeddings, tau)
    sg_flat, idx_flat = _sc_route()(jnp.reshape(fg, (-1,)))
    sg = jnp.reshape(sg_flat, (_NUM_TOK, _NUM_EXPERTS))
    idx = jnp.reshape(idx_flat, (_NUM_TOK, _TOP_K))
    return sg, idx, fg


def kernel(h, W, expert_embeddings, tau):
    return _router(h, W, expert_embeddings, tau)
